# baseline (speedup 1.0000x reference)
import jax
import jax.numpy as jnp
from jax import lax
from jax.experimental import pallas as pl
from jax.experimental.pallas import tpu as pltpu
from jax.experimental.pallas import tpu_sc as plsc

_C = 128
_NSUB = 16
_NCORE = 2
_BSZ = 88
_NBUF = 1
_NROUND = 1
_BLK = 1000


def _sc_agg(h_chunks, src3, dst3, zeros):
    n = h_chunks[0].shape[0]
    nch = len(h_chunks)
    per_core = nch // _NCORE
    tile_rows = zeros.shape[0]
    n_pad = tile_rows * _NSUB
    nbatch, bsz = src3.shape[1], src3.shape[2]

    mesh = plsc.VectorSubcoreMesh(core_axis_name="c", subcore_axis_name="s")

    nbuf = _NBUF
    assert nbatch % nbuf == 0
    nouter = nbatch // nbuf

    nround = _NROUND
    assert nbatch % (nround * nbuf) == 0
    nsub = nbatch // nround
    nsub_outer = nsub // nbuf

    def body(src_hbm, dst_hbm, zeros_hbm, *refs):
        h_refs = refs[:nch]
        out_refs = refs[nch:2 * nch]
        spmem, src_v, dst_v = refs[2 * nch:2 * nch + 3]
        r0 = 2 * nch + 3
        gbufs = refs[r0:r0 + nbuf]
        gsems = refs[r0 + nbuf:r0 + 2 * nbuf]
        ssems = refs[r0 + 2 * nbuf:r0 + 3 * nbuf]
        cid = lax.axis_index("c")
        tid = lax.axis_index("s")
        row0 = tid * tile_rows

        def run_chunk(h_ref, out_ref):
            pltpu.sync_copy(zeros_hbm, spmem.at[pl.ds(row0, tile_rows)])
            plsc.subcore_barrier()

            for rr in range(nround):
                pltpu.sync_copy(src_hbm.at[tid, rr], src_v)
                pltpu.sync_copy(dst_hbm.at[tid, rr], dst_v)
                for j in range(nbuf):
                    pltpu.async_copy(h_ref.at[src_v.at[j]],
                                     gbufs[j], gsems[j])

                def outer(o, carry):
                    for j in range(nbuf):
                        b = o * nbuf + j
                        pltpu.make_async_copy(
                            h_ref.at[src_v.at[b]], gbufs[j], gsems[j]).wait()
                        pltpu.async_copy(gbufs[j], spmem.at[dst_v.at[b]],
                                         ssems[j], add=True)
                    for j in range(nbuf):
                        b = o * nbuf + j
                        pltpu.make_async_copy(
                            gbufs[j], spmem.at[dst_v.at[b]], ssems[j]).wait()

                        @pl.when(o < nsub_outer - 1)
                        def _(j=j, b=b):
                            pltpu.async_copy(h_ref.at[src_v.at[b + nbuf]],
                                             gbufs[j], gsems[j])
                    return carry

                lax.fori_loop(0, nsub_outer, outer, 0)
            plsc.subcore_barrier()
            pltpu.sync_copy(spmem.at[pl.ds(row0, tile_rows)],
                            out_ref.at[pl.ds(row0, tile_rows)])
            plsc.subcore_barrier()

        for core in range(_NCORE):
            def core_work(core=core):
                for j in range(per_core):
                    ch = core * per_core + j
                    run_chunk(h_refs[ch], out_refs[ch])
            pl.when(cid == core)(core_work)

    kern = pl.kernel(
        body,
        out_type=[jax.ShapeDtypeStruct((n_pad, _C), jnp.float32)] * nch,
        scratch_types=[
            pltpu.VMEM_SHARED((n_pad, _C), jnp.float32),
            pltpu.VMEM((nsub, bsz), jnp.int32),
            pltpu.VMEM((nsub, bsz), jnp.int32),
        ] + [pltpu.VMEM((bsz, _C), jnp.float32)] * nbuf
          + [pltpu.SemaphoreType.DMA] * (2 * nbuf),
        mesh=mesh,
    )
    src4 = src3.reshape(_NSUB, nround, nsub, bsz)
    dst4 = dst3.reshape(_NSUB, nround, nsub, bsz)
    outs = kern(src4, dst4, zeros, *h_chunks)
    return outs if isinstance(outs, (tuple, list)) else (outs,)


def _mlp(h_chunks, agg_chunks, w1, b1, w2, b2):
    n = h_chunks[0].shape[0]
    din = w1.shape[0]
    dh = w1.shape[1]
    nch = len(agg_chunks)
    nb = n // _BLK

    def body(*refs):
        h_refs = refs[:nch]
        agg_refs = refs[nch:2 * nch]
        w1_ref, b1_ref, w2_ref, b2_ref, z_ref, ps_ref, pss_ref = refs[2 * nch:]
        i = pl.program_id(0)
        hcat = jnp.concatenate([r[...] for r in h_refs], axis=1)
        agg = jnp.concatenate([r[...] for r in agg_refs], axis=1)
        u = hcat + agg
        z1 = jnp.dot(u, w1_ref[...], preferred_element_type=jnp.float32)
        z1 = jnp.maximum(z1 + b1_ref[...], 0.0)
        z = jnp.dot(z1, w2_ref[...], preferred_element_type=jnp.float32)
        z = jnp.maximum(z + b2_ref[...], 0.0)
        z_ref[...] = z
        s = jnp.sum(z, axis=0, keepdims=True)
        ss = jnp.sum(z * z, axis=0, keepdims=True)

        @pl.when(i == 0)
        def _():
            ps_ref[...] = s
            pss_ref[...] = ss

        @pl.when(i > 0)
        def _():
            ps_ref[...] = ps_ref[...] + s
            pss_ref[...] = pss_ref[...] + ss

    in_specs = [pl.BlockSpec((_BLK, _C), lambda i: (i, 0))] * (2 * nch)
    in_specs += [
        pl.BlockSpec((din, dh), lambda i: (0, 0)),
        pl.BlockSpec((1, dh), lambda i: (0, 0)),
        pl.BlockSpec((dh, dh), lambda i: (0, 0)),
        pl.BlockSpec((1, dh), lambda i: (0, 0)),
    ]
    out_specs = [
        pl.BlockSpec((_BLK, dh), lambda i: (i, 0)),
        pl.BlockSpec((1, dh), lambda i: (0, 0)),
        pl.BlockSpec((1, dh), lambda i: (0, 0)),
    ]
    return pl.pallas_call(
        body,
        grid=(nb,),
        in_specs=in_specs,
        out_specs=out_specs,
        out_shape=[
            jax.ShapeDtypeStruct((n, dh), jnp.float32),
            jax.ShapeDtypeStruct((1, dh), jnp.float32),
            jax.ShapeDtypeStruct((1, dh), jnp.float32),
        ],
    )(*h_chunks, *agg_chunks, w1, b1.reshape(1, -1), w2, b2.reshape(1, -1))


def _bn_apply(z, ps, pss, gamma, beta):
    n, dh = z.shape
    nb = n // _BLK
    nch_out = dh // _C
    inv_n = 1.0 / n

    def body(z_ref, ps_ref, pss_ref, g_ref, b_ref, *chunk_refs):
        mean = ps_ref[...] * inv_n
        var = pss_ref[...] * inv_n - mean * mean
        scale = lax.rsqrt(var + 1e-5) * g_ref[...]
        hv = (z_ref[...] - mean) * scale + b_ref[...]
        for c, cr in enumerate(chunk_refs):
            cr[...] = hv[:, c * _C:(c + 1) * _C]

    in_specs = [
        pl.BlockSpec((_BLK, dh), lambda i: (i, 0)),
        pl.BlockSpec((1, dh), lambda i: (0, 0)),
        pl.BlockSpec((1, dh), lambda i: (0, 0)),
        pl.BlockSpec((1, dh), lambda i: (0, 0)),
        pl.BlockSpec((1, dh), lambda i: (0, 0)),
    ]
    out_specs = [pl.BlockSpec((_BLK, _C), lambda i: (i, 0))] * nch_out
    out_shape = [jax.ShapeDtypeStruct((n, _C), jnp.float32)] * nch_out
    outs = pl.pallas_call(
        body,
        grid=(nb,),
        in_specs=in_specs,
        out_specs=out_specs,
        out_shape=out_shape,
    )(z, ps, pss, gamma.reshape(1, -1), beta.reshape(1, -1))
    return list(outs)


def _pool_head(z, ps, pss, gamma, beta, batch3, w1, b1, w2, b2, g):
    n, dh = z.shape
    nb = n // _BLK
    dout = w2.shape[1]
    inv_n = 1.0 / n

    def body(b_ref, z_ref, ps_ref, pss_ref, ga_ref, be_ref,
             w1_ref, b1_ref, w2_ref, b2_ref, out_ref, acc, cnt):
        i = pl.program_id(0)
        mean = ps_ref[...] * inv_n
        var = pss_ref[...] * inv_n - mean * mean
        scale = lax.rsqrt(var + 1e-5) * ga_ref[...]
        hv = (z_ref[...] - mean) * scale + be_ref[...]
        bvec = b_ref[0, 0, :]
        seg = lax.broadcasted_iota(jnp.int32, (g, _BLK), 0)
        onehot = (bvec[None, :] == seg).astype(jnp.float32)
        part = jnp.dot(onehot, hv, preferred_element_type=jnp.float32)
        c = jnp.sum(onehot, axis=1, keepdims=True)

        @pl.when(i == 0)
        def _():
            acc[...] = part
            cnt[...] = c

        @pl.when(i > 0)
        def _():
            acc[...] = acc[...] + part
            cnt[...] = cnt[...] + c

        @pl.when(i == nb - 1)
        def _():
            pooled = acc[...] / jnp.maximum(cnt[...], 1.0)
            y = jnp.dot(pooled, w1_ref[...], preferred_element_type=jnp.float32)
            y = jnp.maximum(y + b1_ref[...], 0.0)
            y = jnp.dot(y, w2_ref[...], preferred_element_type=jnp.float32)
            y = y + b2_ref[...]
            m = jnp.max(y, axis=-1, keepdims=True)
            lse = jnp.log(jnp.sum(jnp.exp(y - m), axis=-1, keepdims=True)) + m
            out_ref[...] = y - lse

    in_specs = [
        pl.BlockSpec((1, 1, _BLK), lambda i: (i, 0, 0)),
        pl.BlockSpec((_BLK, dh), lambda i: (i, 0)),
        pl.BlockSpec((1, dh), lambda i: (0, 0)),
        pl.BlockSpec((1, dh), lambda i: (0, 0)),
        pl.BlockSpec((1, dh), lambda i: (0, 0)),
        pl.BlockSpec((1, dh), lambda i: (0, 0)),
        pl.BlockSpec((dh, dh), lambda i: (0, 0)),
        pl.BlockSpec((1, dh), lambda i: (0, 0)),
        pl.BlockSpec((dh, dout), lambda i: (0, 0)),
        pl.BlockSpec((1, dout), lambda i: (0, 0)),
    ]
    return pl.pallas_call(
        body,
        grid=(nb,),
        in_specs=in_specs,
        out_specs=pl.BlockSpec((g, dout), lambda i: (0, 0)),
        out_shape=jax.ShapeDtypeStruct((g, dout), jnp.float32),
        scratch_shapes=[
            pltpu.VMEM((g, dh), jnp.float32),
            pltpu.VMEM((g, 1), jnp.float32),
        ],
    )(batch3, z, ps, pss, gamma.reshape(1, -1), beta.reshape(1, -1),
      w1, b1.reshape(1, -1), w2, b2.reshape(1, -1))


def kernel(x, edge_index, batch, params):
    n, d = x.shape
    e = edge_index.shape[1]
    num_layers = sum(1 for k in params if k.startswith("W1_"))
    g = 16

    tile_rows = -(-n // (8 * _NSUB)) * 8
    n_pad = tile_rows * _NSUB
    q = _NBUF * _NROUND
    nbatch_min = -(-e // (_NSUB * _BSZ))
    nbatch = -(-nbatch_min // q) * q
    per_tile = nbatch * _BSZ
    e_pad = per_tile * _NSUB
    assert n_pad > n
    src_p = jnp.concatenate(
        [edge_index[0], jnp.zeros((e_pad - e,), jnp.int32)])
    pad_dst = n + jnp.arange(e_pad - e, dtype=jnp.int32) % (n_pad - n)
    dst_p = jnp.concatenate([edge_index[1], pad_dst])
    src3 = src_p.reshape(_NSUB, nbatch, _BSZ)
    dst3 = dst_p.reshape(_NSUB, nbatch, _BSZ)
    zeros = jnp.zeros((tile_rows, _C), jnp.float32)

    h_chunks = [x[:, c * _C:(c + 1) * _C] for c in range(d // _C)]
    for i in range(num_layers):
        aggs = _sc_agg(h_chunks, src3, dst3, zeros)
        z, ps, pss = _mlp(h_chunks, list(aggs), params[f"W1_{i}"],
                          params[f"b1_{i}"], params[f"W2_{i}"],
                          params[f"b2_{i}"])
        if i < num_layers - 1:
            h_chunks = _bn_apply(z, ps, pss, params[f"gamma_{i}"],
                                 params[f"beta_{i}"])

    batch3 = batch.reshape(n // _BLK, 1, _BLK)
    i = num_layers - 1
    return _pool_head(z, ps, pss, params[f"gamma_{i}"], params[f"beta_{i}"],
                      batch3, params["fc1_W"], params["fc1_b"],
                      params["fc2_W"], params["fc2_b"], g)


# speedup vs baseline: 1.0339x; 1.0339x over previous
import jax
import jax.numpy as jnp
from jax import lax
from jax.experimental import pallas as pl
from jax.experimental.pallas import tpu as pltpu
from jax.experimental.pallas import tpu_sc as plsc

_C = 128
_NSUB = 16
_NCORE = 2
_BSZ = 80
_NBUF = 1
_NROUND = 1
_BLK = 1000


def _sc_agg(h_chunks, src3, dst3, zeros):
    n = h_chunks[0].shape[0]
    nch = len(h_chunks)
    per_core = nch // _NCORE
    tile_rows = zeros.shape[0]
    n_pad = tile_rows * _NSUB
    nbatch, bsz = src3.shape[1], src3.shape[2]

    mesh = plsc.VectorSubcoreMesh(core_axis_name="c", subcore_axis_name="s")

    nbuf = _NBUF
    assert nbatch % nbuf == 0
    nouter = nbatch // nbuf

    nround = _NROUND
    assert nbatch % (nround * nbuf) == 0
    nsub = nbatch // nround
    nsub_outer = nsub // nbuf

    def body(src_hbm, dst_hbm, zeros_hbm, *refs):
        h_refs = refs[:nch]
        out_refs = refs[nch:2 * nch]
        spmem, src_v, dst_v = refs[2 * nch:2 * nch + 3]
        r0 = 2 * nch + 3
        gbufs = refs[r0:r0 + nbuf]
        gsems = refs[r0 + nbuf:r0 + 2 * nbuf]
        ssems = refs[r0 + 2 * nbuf:r0 + 3 * nbuf]
        cid = lax.axis_index("c")
        tid = lax.axis_index("s")
        row0 = tid * tile_rows

        def run_chunk(h_ref, out_ref):
            pltpu.sync_copy(zeros_hbm, spmem.at[pl.ds(row0, tile_rows)])
            plsc.subcore_barrier()

            for rr in range(nround):
                pltpu.sync_copy(src_hbm.at[tid, rr], src_v)
                pltpu.sync_copy(dst_hbm.at[tid, rr], dst_v)
                for j in range(nbuf):
                    pltpu.async_copy(h_ref.at[src_v.at[j]],
                                     gbufs[j], gsems[j])

                def outer(o, carry):
                    for j in range(nbuf):
                        b = o * nbuf + j
                        pltpu.make_async_copy(
                            h_ref.at[src_v.at[b]], gbufs[j], gsems[j]).wait()
                        pltpu.async_copy(gbufs[j], spmem.at[dst_v.at[b]],
                                         ssems[j], add=True)
                    for j in range(nbuf):
                        b = o * nbuf + j
                        pltpu.make_async_copy(
                            gbufs[j], spmem.at[dst_v.at[b]], ssems[j]).wait()

                        @pl.when(o < nsub_outer - 1)
                        def _(j=j, b=b):
                            pltpu.async_copy(h_ref.at[src_v.at[b + nbuf]],
                                             gbufs[j], gsems[j])
                    return carry

                lax.fori_loop(0, nsub_outer, outer, 0)
            plsc.subcore_barrier()
            pltpu.sync_copy(spmem.at[pl.ds(row0, tile_rows)],
                            out_ref.at[pl.ds(row0, tile_rows)])
            plsc.subcore_barrier()

        for core in range(_NCORE):
            def core_work(core=core):
                for j in range(per_core):
                    ch = core * per_core + j
                    run_chunk(h_refs[ch], out_refs[ch])
            pl.when(cid == core)(core_work)

    kern = pl.kernel(
        body,
        out_type=[jax.ShapeDtypeStruct((n_pad, _C), jnp.float32)] * nch,
        scratch_types=[
            pltpu.VMEM_SHARED((n_pad, _C), jnp.float32),
            pltpu.VMEM((nsub, bsz), jnp.int32),
            pltpu.VMEM((nsub, bsz), jnp.int32),
        ] + [pltpu.VMEM((bsz, _C), jnp.float32)] * nbuf
          + [pltpu.SemaphoreType.DMA] * (2 * nbuf),
        mesh=mesh,
    )
    src4 = src3.reshape(_NSUB, nround, nsub, bsz)
    dst4 = dst3.reshape(_NSUB, nround, nsub, bsz)
    outs = kern(src4, dst4, zeros, *h_chunks)
    return outs if isinstance(outs, (tuple, list)) else (outs,)


def _mlp(h_chunks, agg_chunks, w1, b1, w2, b2):
    n = h_chunks[0].shape[0]
    din = w1.shape[0]
    dh = w1.shape[1]
    nch = len(agg_chunks)
    nb = n // _BLK

    def body(*refs):
        h_refs = refs[:nch]
        agg_refs = refs[nch:2 * nch]
        w1_ref, b1_ref, w2_ref, b2_ref, z_ref, ps_ref, pss_ref = refs[2 * nch:]
        i = pl.program_id(0)
        hcat = jnp.concatenate([r[...] for r in h_refs], axis=1)
        agg = jnp.concatenate([r[...] for r in agg_refs], axis=1)
        u = hcat + agg
        z1 = jnp.dot(u, w1_ref[...], preferred_element_type=jnp.float32)
        z1 = jnp.maximum(z1 + b1_ref[...], 0.0)
        z = jnp.dot(z1, w2_ref[...], preferred_element_type=jnp.float32)
        z = jnp.maximum(z + b2_ref[...], 0.0)
        z_ref[...] = z
        s = jnp.sum(z, axis=0, keepdims=True)
        ss = jnp.sum(z * z, axis=0, keepdims=True)

        @pl.when(i == 0)
        def _():
            ps_ref[...] = s
            pss_ref[...] = ss

        @pl.when(i > 0)
        def _():
            ps_ref[...] = ps_ref[...] + s
            pss_ref[...] = pss_ref[...] + ss

    in_specs = [pl.BlockSpec((_BLK, _C), lambda i: (i, 0))] * (2 * nch)
    in_specs += [
        pl.BlockSpec((din, dh), lambda i: (0, 0)),
        pl.BlockSpec((1, dh), lambda i: (0, 0)),
        pl.BlockSpec((dh, dh), lambda i: (0, 0)),
        pl.BlockSpec((1, dh), lambda i: (0, 0)),
    ]
    out_specs = [
        pl.BlockSpec((_BLK, dh), lambda i: (i, 0)),
        pl.BlockSpec((1, dh), lambda i: (0, 0)),
        pl.BlockSpec((1, dh), lambda i: (0, 0)),
    ]
    return pl.pallas_call(
        body,
        grid=(nb,),
        in_specs=in_specs,
        out_specs=out_specs,
        out_shape=[
            jax.ShapeDtypeStruct((n, dh), jnp.float32),
            jax.ShapeDtypeStruct((1, dh), jnp.float32),
            jax.ShapeDtypeStruct((1, dh), jnp.float32),
        ],
    )(*h_chunks, *agg_chunks, w1, b1.reshape(1, -1), w2, b2.reshape(1, -1))


def _bn_apply(z, ps, pss, gamma, beta):
    n, dh = z.shape
    nb = n // _BLK
    nch_out = dh // _C
    inv_n = 1.0 / n

    def body(z_ref, ps_ref, pss_ref, g_ref, b_ref, *chunk_refs):
        mean = ps_ref[...] * inv_n
        var = pss_ref[...] * inv_n - mean * mean
        scale = lax.rsqrt(var + 1e-5) * g_ref[...]
        hv = (z_ref[...] - mean) * scale + b_ref[...]
        for c, cr in enumerate(chunk_refs):
            cr[...] = hv[:, c * _C:(c + 1) * _C]

    in_specs = [
        pl.BlockSpec((_BLK, dh), lambda i: (i, 0)),
        pl.BlockSpec((1, dh), lambda i: (0, 0)),
        pl.BlockSpec((1, dh), lambda i: (0, 0)),
        pl.BlockSpec((1, dh), lambda i: (0, 0)),
        pl.BlockSpec((1, dh), lambda i: (0, 0)),
    ]
    out_specs = [pl.BlockSpec((_BLK, _C), lambda i: (i, 0))] * nch_out
    out_shape = [jax.ShapeDtypeStruct((n, _C), jnp.float32)] * nch_out
    outs = pl.pallas_call(
        body,
        grid=(nb,),
        in_specs=in_specs,
        out_specs=out_specs,
        out_shape=out_shape,
    )(z, ps, pss, gamma.reshape(1, -1), beta.reshape(1, -1))
    return list(outs)


def _pool_head(z, ps, pss, gamma, beta, batch3, w1, b1, w2, b2, g):
    n, dh = z.shape
    nb = n // _BLK
    dout = w2.shape[1]
    inv_n = 1.0 / n

    def body(b_ref, z_ref, ps_ref, pss_ref, ga_ref, be_ref,
             w1_ref, b1_ref, w2_ref, b2_ref, out_ref, acc, cnt):
        i = pl.program_id(0)
        mean = ps_ref[...] * inv_n
        var = pss_ref[...] * inv_n - mean * mean
        scale = lax.rsqrt(var + 1e-5) * ga_ref[...]
        hv = (z_ref[...] - mean) * scale + be_ref[...]
        bvec = b_ref[0, 0, :]
        seg = lax.broadcasted_iota(jnp.int32, (g, _BLK), 0)
        onehot = (bvec[None, :] == seg).astype(jnp.float32)
        part = jnp.dot(onehot, hv, preferred_element_type=jnp.float32)
        c = jnp.sum(onehot, axis=1, keepdims=True)

        @pl.when(i == 0)
        def _():
            acc[...] = part
            cnt[...] = c

        @pl.when(i > 0)
        def _():
            acc[...] = acc[...] + part
            cnt[...] = cnt[...] + c

        @pl.when(i == nb - 1)
        def _():
            pooled = acc[...] / jnp.maximum(cnt[...], 1.0)
            y = jnp.dot(pooled, w1_ref[...], preferred_element_type=jnp.float32)
            y = jnp.maximum(y + b1_ref[...], 0.0)
            y = jnp.dot(y, w2_ref[...], preferred_element_type=jnp.float32)
            y = y + b2_ref[...]
            m = jnp.max(y, axis=-1, keepdims=True)
            lse = jnp.log(jnp.sum(jnp.exp(y - m), axis=-1, keepdims=True)) + m
            out_ref[...] = y - lse

    in_specs = [
        pl.BlockSpec((1, 1, _BLK), lambda i: (i, 0, 0)),
        pl.BlockSpec((_BLK, dh), lambda i: (i, 0)),
        pl.BlockSpec((1, dh), lambda i: (0, 0)),
        pl.BlockSpec((1, dh), lambda i: (0, 0)),
        pl.BlockSpec((1, dh), lambda i: (0, 0)),
        pl.BlockSpec((1, dh), lambda i: (0, 0)),
        pl.BlockSpec((dh, dh), lambda i: (0, 0)),
        pl.BlockSpec((1, dh), lambda i: (0, 0)),
        pl.BlockSpec((dh, dout), lambda i: (0, 0)),
        pl.BlockSpec((1, dout), lambda i: (0, 0)),
    ]
    return pl.pallas_call(
        body,
        grid=(nb,),
        in_specs=in_specs,
        out_specs=pl.BlockSpec((g, dout), lambda i: (0, 0)),
        out_shape=jax.ShapeDtypeStruct((g, dout), jnp.float32),
        scratch_shapes=[
            pltpu.VMEM((g, dh), jnp.float32),
            pltpu.VMEM((g, 1), jnp.float32),
        ],
    )(batch3, z, ps, pss, gamma.reshape(1, -1), beta.reshape(1, -1),
      w1, b1.reshape(1, -1), w2, b2.reshape(1, -1))


def kernel(x, edge_index, batch, params):
    n, d = x.shape
    e = edge_index.shape[1]
    num_layers = sum(1 for k in params if k.startswith("W1_"))
    g = 16

    tile_rows = -(-n // (8 * _NSUB)) * 8
    n_pad = tile_rows * _NSUB
    q = _NBUF * _NROUND
    nbatch_min = -(-e // (_NSUB * _BSZ))
    nbatch = -(-nbatch_min // q) * q
    per_tile = nbatch * _BSZ
    e_pad = per_tile * _NSUB
    assert n_pad > n
    src_p = jnp.concatenate(
        [edge_index[0], jnp.zeros((e_pad - e,), jnp.int32)])
    pad_dst = n + jnp.arange(e_pad - e, dtype=jnp.int32) % (n_pad - n)
    dst_p = jnp.concatenate([edge_index[1], pad_dst])
    src3 = src_p.reshape(_NSUB, nbatch, _BSZ)
    dst3 = dst_p.reshape(_NSUB, nbatch, _BSZ)
    zeros = jnp.zeros((tile_rows, _C), jnp.float32)

    h_chunks = [x[:, c * _C:(c + 1) * _C] for c in range(d // _C)]
    for i in range(num_layers):
        aggs = _sc_agg(h_chunks, src3, dst3, zeros)
        z, ps, pss = _mlp(h_chunks, list(aggs), params[f"W1_{i}"],
                          params[f"b1_{i}"], params[f"W2_{i}"],
                          params[f"b2_{i}"])
        if i < num_layers - 1:
            h_chunks = _bn_apply(z, ps, pss, params[f"gamma_{i}"],
                                 params[f"beta_{i}"])

    batch3 = batch.reshape(n // _BLK, 1, _BLK)
    i = num_layers - 1
    return _pool_head(z, ps, pss, params[f"gamma_{i}"], params[f"beta_{i}"],
                      batch3, params["fc1_W"], params["fc1_b"],
                      params["fc2_W"], params["fc2_b"], g)
